# baseline (speedup 1.0000x reference)
import functools

import jax
import jax.numpy as jnp
from jax import lax
from jax.experimental import pallas as pl
from jax.experimental.pallas import tpu as pltpu
from jax.experimental.pallas import tpu_sc as plsc

_F = 128
_CUTOFF = 2.0


def _silu(x):
    return x * jax.nn.sigmoid(x)


def _dotT(a, b):
    return jax.lax.dot_general(a, b, (((1,), (1,)), ((), ())),
                               preferred_element_type=jnp.float32)


def _sc_gather_build(n, f, n_chunk):
    mesh = plsc.VectorSubcoreMesh(core_axis_name="c", subcore_axis_name="s")

    @functools.partial(
        pl.kernel, mesh=mesh,
        out_type=jax.ShapeDtypeStruct((n, f), jnp.float32),
        scratch_types=[
            pltpu.VMEM((n_chunk,), jnp.int32),
            pltpu.VMEM((n_chunk, f), jnp.float32),
            pltpu.SemaphoreType.DMA,
        ],
    )
    def sc_gather(table_hbm, idx_hbm, out_hbm, idx_v, rows_v, sem):
        wid = lax.axis_index("s") * 2 + lax.axis_index("c")
        base = wid * n_chunk
        pltpu.sync_copy(idx_hbm.at[pl.ds(base, n_chunk)], idx_v)
        pltpu.async_copy(table_hbm.at[idx_v], rows_v, sem).wait()
        pltpu.sync_copy(rows_v, out_hbm.at[pl.ds(base, n_chunk)])

    return sc_gather


def _table_kernel(emb_ref, w1_ref, b1_ref, w2_ref, b2_ref, rbfb_ref, gt_ref):
    emb = emb_ref[...]
    h1 = _silu(_dotT(emb, w1_ref[...]) + b1_ref[0, :])
    phi = _dotT(h1, w2_ref[...]) + b2_ref[0, :]
    rbfb = rbfb_ref[0, :]
    sp1 = phi[:, :_F] * rbfb[None, :_F]
    sp3 = phi[:, 2 * _F:] * rbfb[None, 2 * _F:]
    gt_ref[...] = jnp.concatenate([emb, sp1, sp3], axis=1)


def _mask_tiles(posT_ref, sq_ref, batch_ref, r, ti, t, tj, want_dist):
    pi = posT_ref[:, pl.ds(r * ti, ti)]
    pj = posT_ref[:, pl.ds(t * tj, tj)]
    dot = jax.lax.dot_general(pi, pj, (((0,), (0,)), ((), ())),
                              preferred_element_type=jnp.float32)
    sq_i = sq_ref[0, pl.ds(r * ti, ti)][:, None]
    sq_j = sq_ref[0, pl.ds(t * tj, tj)][None, :]
    d2 = sq_i + sq_j - 2.0 * dot
    dist_m = jnp.sqrt(jnp.maximum(d2, 0.0))
    b_i = batch_ref[0, pl.ds(r * ti, ti)][:, None]
    b_j = batch_ref[0, pl.ds(t * tj, tj)][None, :]
    adj = (dist_m <= _CUTOFF) & (b_i == b_j)
    row_ids = r * ti + jax.lax.broadcasted_iota(jnp.int32, (ti, tj), 0)
    col_ids = t * tj + jax.lax.broadcasted_iota(jnp.int32, (ti, tj), 1)
    a = adj.astype(jnp.float32) - (row_ids == col_ids).astype(jnp.float32)
    if not want_dist:
        return a, None
    rel2 = ((pi[0][:, None] - pj[0][None, :]) ** 2
            + (pi[1][:, None] - pj[1][None, :]) ** 2
            + (pi[2][:, None] - pj[2][None, :]) ** 2)
    return a, a * jnp.sqrt(rel2)


def _agg1_kernel(lo_ref, hi_ref, posT_ref, sq_ref, batch_ref, g_ref,
                 uw_ref, vw_ref, up1_ref, up1b_ref, up2_ref, up2b_ref,
                 s1_ref, ds2_ref, acc_ds, acc_m, *, tj):
    ti = s1_ref.shape[0]
    r = pl.program_id(0)
    acc_ds[...] = jnp.zeros_like(acc_ds)
    acc_m[...] = jnp.zeros_like(acc_m)

    def body(t, _):
        a, b = _mask_tiles(posT_ref, sq_ref, batch_ref, r, ti, t, tj, True)
        sp1 = g_ref[pl.ds(t * tj, tj), _F:2 * _F]
        sp3 = g_ref[pl.ds(t * tj, tj), 2 * _F:]
        pxj = posT_ref[0, pl.ds(t * tj, tj)][None, :]
        pyj = posT_ref[1, pl.ds(t * tj, tj)][None, :]
        pzj = posT_ref[2, pl.ds(t * tj, tj)][None, :]
        acc_ds[...] += jnp.dot(a, sp1, preferred_element_type=jnp.float32)
        acc_m[:, :_F] += jnp.dot(b, sp3, preferred_element_type=jnp.float32)
        acc_m[:, _F:2 * _F] += jnp.dot(b * pxj, sp3,
                                       preferred_element_type=jnp.float32)
        acc_m[:, 2 * _F:3 * _F] += jnp.dot(b * pyj, sp3,
                                           preferred_element_type=jnp.float32)
        acc_m[:, 3 * _F:] += jnp.dot(b * pzj, sp3,
                                     preferred_element_type=jnp.float32)
        return 0

    jax.lax.fori_loop(lo_ref[r], hi_ref[r], body, 0)

    ds = acc_ds[...]
    m = acc_m[...]
    px = posT_ref[0, pl.ds(r * ti, ti)][:, None]
    py = posT_ref[1, pl.ds(r * ti, ti)][:, None]
    pz = posT_ref[2, pl.ds(r * ti, ti)][:, None]
    m3 = m[:, :_F]
    vx = px * m3 - m[:, _F:2 * _F]
    vy = py * m3 - m[:, 2 * _F:3 * _F]
    vz = pz * m3 - m[:, 3 * _F:]
    s1 = g_ref[pl.ds(r * ti, ti), :_F] + ds
    s1_ref[...] = s1
    uw = uw_ref[...]
    vw = vw_ref[...]
    ux, uy, uz = _dotT(vx, uw), _dotT(vy, uw), _dotT(vz, uw)
    vvx, vvy, vvz = _dotT(vx, vw), _dotT(vy, vw), _dotT(vz, vw)
    vnorm = jnp.sqrt(vvx * vvx + vvy * vvy + vvz * vvz)
    stack = jnp.concatenate([s1, vnorm], axis=1)
    h = _silu(_dotT(stack, up1_ref[...]) + up1b_ref[0, :])
    split = _dotT(h, up2_ref[...]) + up2b_ref[0, :]
    uv = ux * vvx + uy * vvy + uz * vvz
    ds2_ref[...] = split[:, _F:2 * _F] + uv * split[:, 2 * _F:]


def _agg2_kernel(lo_ref, hi_ref, posT_ref, sq_ref, batch_ref, ds2_ref, s1_ref,
                 o1_ref, o1b_ref, o2_ref, o2b_ref, y_ref, acc, *, tj, n_graphs):
    ti = s1_ref.shape[0]
    r = pl.program_id(0)
    acc[...] = jnp.zeros_like(acc)

    @pl.when(r == 0)
    def _():
        y_ref[...] = jnp.zeros_like(y_ref)

    def body(t, _):
        a, _unused = _mask_tiles(posT_ref, sq_ref, batch_ref, r, ti, t, tj,
                                 False)
        d = ds2_ref[pl.ds(t * tj, tj), :]
        acc[...] += jnp.dot(a, d, preferred_element_type=jnp.float32)
        return 0

    jax.lax.fori_loop(lo_ref[r], hi_ref[r], body, 0)

    s2 = s1_ref[...] + acc[...]
    h = _silu(_dotT(s2, o1_ref[...]) + o1b_ref[0, :])
    hw = h * o2_ref[0, :][None, :]
    bt = batch_ref[0, pl.ds(r * ti, ti)][:, None]
    oh = (bt == jax.lax.broadcasted_iota(jnp.int32, (ti, n_graphs), 1))
    ohf = oh.astype(jnp.float32)
    m1 = jax.lax.dot_general(ohf, hw, (((0,), (0,)), ((), ())),
                             preferred_element_type=jnp.float32)
    contrib = jnp.sum(m1, axis=1) + o2b_ref[0, 0] * jnp.sum(ohf, axis=0)
    y_ref[...] += contrib[None, :]


def _full_spec(shape):
    nd = len(shape)
    return pl.BlockSpec(shape, lambda r, *_: (0,) * nd)


def kernel(z, pos, batch, emb_table, phi1_w, phi1_b, phi2_w, phi2_b, rbfW_w,
           rbfW_b, U_w, V_w, up1_w, up1_b, up2_w, up2_b, out1_w, out1_b,
           out2_w, out2_b):
    n = z.shape[0]
    n_graphs = 128
    ti = min(512, n)
    tj = min(512, n)
    nrt = n // ti

    batch2 = batch.reshape(1, n).astype(jnp.int32)
    posT = pos.T
    sq = jnp.sum(pos * pos, axis=1).reshape(1, n)

    br = batch.reshape(nrt, ti)
    bmin = br[:, 0]
    bmax = br[:, -1]
    starts = jnp.sum(batch[None, :] < bmin[:, None], axis=1)
    ends = jnp.sum(batch[None, :] <= bmax[:, None], axis=1)
    lo = (starts // tj).astype(jnp.int32)
    hi = ((ends + tj - 1) // tj).astype(jnp.int32)

    num_atoms = emb_table.shape[0]
    gtable = pl.pallas_call(
        _table_kernel,
        grid=(1,),
        in_specs=[
            _full_spec(emb_table.shape),
            _full_spec(phi1_w.shape),
            _full_spec((1, _F)),
            _full_spec(phi2_w.shape),
            _full_spec((1, 3 * _F)),
            _full_spec((1, 3 * _F)),
        ],
        out_specs=_full_spec((num_atoms, 3 * _F)),
        out_shape=jax.ShapeDtypeStruct((num_atoms, 3 * _F), jnp.float32),
    )(emb_table, phi1_w, phi1_b.reshape(1, _F), phi2_w,
      phi2_b.reshape(1, 3 * _F), rbfW_b.reshape(1, 3 * _F))

    g = _sc_gather_build(n, 3 * _F, n // 32)(gtable, z.astype(jnp.int32))

    grid_spec1 = pltpu.PrefetchScalarGridSpec(
        num_scalar_prefetch=2,
        grid=(nrt,),
        in_specs=[
            _full_spec((3, n)),
            _full_spec((1, n)),
            _full_spec((1, n)),
            _full_spec((n, 3 * _F)),
            _full_spec(U_w.shape),
            _full_spec(V_w.shape),
            _full_spec(up1_w.shape),
            _full_spec((1, _F)),
            _full_spec(up2_w.shape),
            _full_spec((1, 3 * _F)),
        ],
        out_specs=[
            pl.BlockSpec((ti, _F), lambda r, *_: (r, 0)),
            pl.BlockSpec((ti, _F), lambda r, *_: (r, 0)),
        ],
        scratch_shapes=[
            pltpu.VMEM((ti, _F), jnp.float32),
            pltpu.VMEM((ti, 4 * _F), jnp.float32),
        ],
    )
    s1, ds2 = pl.pallas_call(
        functools.partial(_agg1_kernel, tj=tj),
        grid_spec=grid_spec1,
        out_shape=[
            jax.ShapeDtypeStruct((n, _F), jnp.float32),
            jax.ShapeDtypeStruct((n, _F), jnp.float32),
        ],
    )(lo, hi, posT, sq, batch2, g, U_w, V_w, up1_w,
      up1_b.reshape(1, _F), up2_w, up2_b.reshape(1, 3 * _F))

    grid_spec2 = pltpu.PrefetchScalarGridSpec(
        num_scalar_prefetch=2,
        grid=(nrt,),
        in_specs=[
            _full_spec((3, n)),
            _full_spec((1, n)),
            _full_spec((1, n)),
            _full_spec((n, _F)),
            pl.BlockSpec((ti, _F), lambda r, *_: (r, 0)),
            _full_spec(out1_w.shape),
            _full_spec((1, _F)),
            _full_spec(out2_w.shape),
            _full_spec((1, 1)),
        ],
        out_specs=pl.BlockSpec((1, n_graphs), lambda r, *_: (0, 0)),
        scratch_shapes=[pltpu.VMEM((ti, _F), jnp.float32)],
    )
    y = pl.pallas_call(
        functools.partial(_agg2_kernel, tj=tj, n_graphs=n_graphs),
        grid_spec=grid_spec2,
        out_shape=jax.ShapeDtypeStruct((1, n_graphs), jnp.float32),
    )(lo, hi, posT, sq, batch2, ds2, s1, out1_w, out1_b.reshape(1, _F),
      out2_w, out2_b.reshape(1, 1))

    return y.reshape(n_graphs, 1)


# speedup vs baseline: 1.4918x; 1.4918x over previous
import functools

import jax
import jax.numpy as jnp
from jax import lax
from jax.experimental import pallas as pl
from jax.experimental.pallas import tpu as pltpu
from jax.experimental.pallas import tpu_sc as plsc

_F = 128
_CUTOFF = 2.0


def _silu(x):
    return x * jax.nn.sigmoid(x)


def _dotT(a, b):
    return jax.lax.dot_general(a, b, (((1,), (1,)), ((), ())),
                               preferred_element_type=jnp.float32)


def _sc_gather_build(n, f, n_chunk):
    mesh = plsc.VectorSubcoreMesh(core_axis_name="c", subcore_axis_name="s")

    @functools.partial(
        pl.kernel, mesh=mesh,
        out_type=jax.ShapeDtypeStruct((n, f), jnp.float32),
        scratch_types=[
            pltpu.VMEM((n_chunk,), jnp.int32),
            pltpu.VMEM((n_chunk, f), jnp.float32),
            pltpu.SemaphoreType.DMA,
        ],
    )
    def sc_gather(table_hbm, idx_hbm, out_hbm, idx_v, rows_v, sem):
        wid = lax.axis_index("s") * 2 + lax.axis_index("c")
        base = wid * n_chunk
        pltpu.sync_copy(idx_hbm.at[pl.ds(base, n_chunk)], idx_v)
        pltpu.async_copy(table_hbm.at[idx_v], rows_v, sem).wait()
        pltpu.sync_copy(rows_v, out_hbm.at[pl.ds(base, n_chunk)])

    return sc_gather


def _table_kernel(emb_ref, w1_ref, b1_ref, w2_ref, b2_ref, rbfb_ref, gt_ref):
    emb = emb_ref[...]
    h1 = _silu(_dotT(emb, w1_ref[...]) + b1_ref[0, :])
    phi = _dotT(h1, w2_ref[...]) + b2_ref[0, :]
    rbfb = rbfb_ref[0, :]
    sp1 = phi[:, :_F] * rbfb[None, :_F]
    sp3 = phi[:, 2 * _F:] * rbfb[None, 2 * _F:]
    gt_ref[...] = jnp.concatenate([sp1, sp3], axis=1)


def _mask_tiles(posT_ref, sq_ref, batch_ref, r, ti, cs, base, tj, want_dist):
    pi = posT_ref[:, pl.ds(r * ti, ti)]
    pj = posT_ref[:, pl.ds(cs, tj)]
    dot = jax.lax.dot_general(pi, pj, (((0,), (0,)), ((), ())),
                              preferred_element_type=jnp.float32)
    sq_i = sq_ref[0, pl.ds(r * ti, ti)][:, None]
    sq_j = sq_ref[0, pl.ds(cs, tj)][None, :]
    d2 = sq_i + sq_j - 2.0 * dot
    dist_m = jnp.sqrt(jnp.maximum(d2, 0.0))
    b_i = batch_ref[0, pl.ds(r * ti, ti)][:, None]
    b_j = batch_ref[0, pl.ds(cs, tj)][None, :]
    row_ids = r * ti + jax.lax.broadcasted_iota(jnp.int32, (ti, tj), 0)
    col_ids = cs + jax.lax.broadcasted_iota(jnp.int32, (ti, tj), 1)
    adj = (dist_m <= _CUTOFF) & (b_i == b_j) & (col_ids >= base)
    a = adj.astype(jnp.float32) - ((row_ids == col_ids)
                                   & (col_ids >= base)).astype(jnp.float32)
    if not want_dist:
        return a, None
    return a, a * dist_m


def _agg1_kernel(ws_ref, nw_ref, posT_ref, sq_ref, batch_ref, z_ref, emb_ref,
                 g_ref, uw_ref, vw_ref, up1_ref, up1b_ref, up2_ref, up2b_ref,
                 s1_ref, ds2_ref, acc_ds, acc_m, *, tj):
    ti = s1_ref.shape[0]
    n = sq_ref.shape[1]
    r = pl.program_id(0)
    acc_ds[...] = jnp.zeros_like(acc_ds)
    acc_m[...] = jnp.zeros_like(acc_m)

    def body(w, _):
        base = ws_ref[r] + w * tj
        cs = pl.multiple_of(jnp.minimum(base, n - tj), 128)
        a, b = _mask_tiles(posT_ref, sq_ref, batch_ref, r, ti, cs, base, tj,
                           True)
        sp1 = g_ref[pl.ds(cs, tj), :_F]
        sp3 = g_ref[pl.ds(cs, tj), _F:]
        pxj = posT_ref[0, pl.ds(cs, tj)][None, :]
        pyj = posT_ref[1, pl.ds(cs, tj)][None, :]
        pzj = posT_ref[2, pl.ds(cs, tj)][None, :]
        acc_ds[...] += jnp.dot(a, sp1, preferred_element_type=jnp.float32)
        acc_m[:, :_F] += jnp.dot(b, sp3, preferred_element_type=jnp.float32)
        acc_m[:, _F:2 * _F] += jnp.dot(b * pxj, sp3,
                                       preferred_element_type=jnp.float32)
        acc_m[:, 2 * _F:3 * _F] += jnp.dot(b * pyj, sp3,
                                           preferred_element_type=jnp.float32)
        acc_m[:, 3 * _F:] += jnp.dot(b * pzj, sp3,
                                     preferred_element_type=jnp.float32)
        return 0

    jax.lax.fori_loop(0, nw_ref[r], body, 0)

    ds = acc_ds[...]
    m = acc_m[...]
    px = posT_ref[0, pl.ds(r * ti, ti)][:, None]
    py = posT_ref[1, pl.ds(r * ti, ti)][:, None]
    pz = posT_ref[2, pl.ds(r * ti, ti)][:, None]
    m3 = m[:, :_F]
    vx = px * m3 - m[:, _F:2 * _F]
    vy = py * m3 - m[:, 2 * _F:3 * _F]
    vz = pz * m3 - m[:, 3 * _F:]
    z = z_ref[0, pl.ds(r * ti, ti)]
    num_atoms = emb_ref.shape[0]
    oh = (z[:, None] == jax.lax.broadcasted_iota(jnp.int32, (ti, num_atoms), 1))
    s0 = jnp.dot(oh.astype(jnp.float32), emb_ref[...],
                 preferred_element_type=jnp.float32)
    s1 = s0 + ds
    s1_ref[...] = s1
    uw = uw_ref[...]
    vw = vw_ref[...]
    ux, uy, uz = _dotT(vx, uw), _dotT(vy, uw), _dotT(vz, uw)
    vvx, vvy, vvz = _dotT(vx, vw), _dotT(vy, vw), _dotT(vz, vw)
    vnorm = jnp.sqrt(vvx * vvx + vvy * vvy + vvz * vvz)
    stack = jnp.concatenate([s1, vnorm], axis=1)
    h = _silu(_dotT(stack, up1_ref[...]) + up1b_ref[0, :])
    split = _dotT(h, up2_ref[...]) + up2b_ref[0, :]
    uv = ux * vvx + uy * vvy + uz * vvz
    ds2_ref[...] = split[:, _F:2 * _F] + uv * split[:, 2 * _F:]


def _agg2_kernel(ws_ref, nw_ref, posT_ref, sq_ref, batch_ref, ds2_ref, s1_ref,
                 o1_ref, o1b_ref, o2_ref, o2b_ref, y_ref, acc, *, tj, n_graphs):
    ti = s1_ref.shape[0]
    n = sq_ref.shape[1]
    r = pl.program_id(0)
    acc[...] = jnp.zeros_like(acc)

    @pl.when(r == 0)
    def _():
        y_ref[...] = jnp.zeros_like(y_ref)

    def body(w, _):
        base = ws_ref[r] + w * tj
        cs = pl.multiple_of(jnp.minimum(base, n - tj), 128)
        a, _unused = _mask_tiles(posT_ref, sq_ref, batch_ref, r, ti, cs, base,
                                 tj, False)
        d = ds2_ref[pl.ds(cs, tj), :]
        acc[...] += jnp.dot(a, d, preferred_element_type=jnp.float32)
        return 0

    jax.lax.fori_loop(0, nw_ref[r], body, 0)

    s2 = s1_ref[...] + acc[...]
    h = _silu(_dotT(s2, o1_ref[...]) + o1b_ref[0, :])
    hw = h * o2_ref[0, :][None, :]
    bt = batch_ref[0, pl.ds(r * ti, ti)][:, None]
    oh = (bt == jax.lax.broadcasted_iota(jnp.int32, (ti, n_graphs), 1))
    ohf = oh.astype(jnp.float32)
    m1 = jax.lax.dot_general(ohf, hw, (((0,), (0,)), ((), ())),
                             preferred_element_type=jnp.float32)
    contrib = jnp.sum(m1, axis=1) + o2b_ref[0, 0] * jnp.sum(ohf, axis=0)
    y_ref[...] += contrib[None, :]


def _full_spec(shape):
    nd = len(shape)
    return pl.BlockSpec(shape, lambda r, *_: (0,) * nd)


def kernel(z, pos, batch, emb_table, phi1_w, phi1_b, phi2_w, phi2_b, rbfW_w,
           rbfW_b, U_w, V_w, up1_w, up1_b, up2_w, up2_b, out1_w, out1_b,
           out2_w, out2_b):
    n = z.shape[0]
    n_graphs = 128
    ti = min(512, n)
    tj = min(768, n)
    nrt = n // ti

    z2 = z.reshape(1, n).astype(jnp.int32)
    batch2 = batch.reshape(1, n).astype(jnp.int32)
    posT = pos.T
    sq = jnp.sum(pos * pos, axis=1).reshape(1, n)

    br = batch.reshape(nrt, ti)
    bmin = br[:, 0]
    bmax = br[:, -1]
    starts = jnp.sum(batch[None, :] < bmin[:, None], axis=1)
    ends = jnp.sum(batch[None, :] <= bmax[:, None], axis=1)
    ws = ((starts // 128) * 128).astype(jnp.int32)
    nw = ((ends - ws + tj - 1) // tj).astype(jnp.int32)

    num_atoms = emb_table.shape[0]
    gtable = pl.pallas_call(
        _table_kernel,
        grid=(1,),
        in_specs=[
            _full_spec(emb_table.shape),
            _full_spec(phi1_w.shape),
            _full_spec((1, _F)),
            _full_spec(phi2_w.shape),
            _full_spec((1, 3 * _F)),
            _full_spec((1, 3 * _F)),
        ],
        out_specs=_full_spec((num_atoms, 2 * _F)),
        out_shape=jax.ShapeDtypeStruct((num_atoms, 2 * _F), jnp.float32),
    )(emb_table, phi1_w, phi1_b.reshape(1, _F), phi2_w,
      phi2_b.reshape(1, 3 * _F), rbfW_b.reshape(1, 3 * _F))

    g = _sc_gather_build(n, 2 * _F, n // 32)(gtable, z.astype(jnp.int32))

    grid_spec1 = pltpu.PrefetchScalarGridSpec(
        num_scalar_prefetch=2,
        grid=(nrt,),
        in_specs=[
            _full_spec((3, n)),
            _full_spec((1, n)),
            _full_spec((1, n)),
            _full_spec((1, n)),
            _full_spec(emb_table.shape),
            _full_spec((n, 2 * _F)),
            _full_spec(U_w.shape),
            _full_spec(V_w.shape),
            _full_spec(up1_w.shape),
            _full_spec((1, _F)),
            _full_spec(up2_w.shape),
            _full_spec((1, 3 * _F)),
        ],
        out_specs=[
            pl.BlockSpec((ti, _F), lambda r, *_: (r, 0)),
            pl.BlockSpec((ti, _F), lambda r, *_: (r, 0)),
        ],
        scratch_shapes=[
            pltpu.VMEM((ti, _F), jnp.float32),
            pltpu.VMEM((ti, 4 * _F), jnp.float32),
        ],
    )
    s1, ds2 = pl.pallas_call(
        functools.partial(_agg1_kernel, tj=tj),
        grid_spec=grid_spec1,
        out_shape=[
            jax.ShapeDtypeStruct((n, _F), jnp.float32),
            jax.ShapeDtypeStruct((n, _F), jnp.float32),
        ],
    )(ws, nw, posT, sq, batch2, z2, emb_table, g, U_w, V_w, up1_w,
      up1_b.reshape(1, _F), up2_w, up2_b.reshape(1, 3 * _F))

    grid_spec2 = pltpu.PrefetchScalarGridSpec(
        num_scalar_prefetch=2,
        grid=(nrt,),
        in_specs=[
            _full_spec((3, n)),
            _full_spec((1, n)),
            _full_spec((1, n)),
            _full_spec((n, _F)),
            pl.BlockSpec((ti, _F), lambda r, *_: (r, 0)),
            _full_spec(out1_w.shape),
            _full_spec((1, _F)),
            _full_spec(out2_w.shape),
            _full_spec((1, 1)),
        ],
        out_specs=pl.BlockSpec((1, n_graphs), lambda r, *_: (0, 0)),
        scratch_shapes=[pltpu.VMEM((ti, _F), jnp.float32)],
    )
    y = pl.pallas_call(
        functools.partial(_agg2_kernel, tj=tj, n_graphs=n_graphs),
        grid_spec=grid_spec2,
        out_shape=jax.ShapeDtypeStruct((1, n_graphs), jnp.float32),
    )(ws, nw, posT, sq, batch2, ds2, s1, out1_w, out1_b.reshape(1, _F),
      out2_w, out2_b.reshape(1, 1))

    return y.reshape(n_graphs, 1)
